# baseline (speedup 1.0000x reference)
import functools
import math

import jax
import jax.numpy as jnp
from jax import lax
from jax.experimental import pallas as pl
from jax.experimental.pallas import tpu as pltpu
from jax.experimental.pallas import tpu_sc as plsc

N_ITEMS = 1000000
D_DIM = 32
S_SAMPLES = 1024

NUM_WORKERS = 32
ROWS_PER_WORKER = S_SAMPLES // NUM_WORKERS
LANES = 16

BLOCK_ITEMS = 8192
NUM_BLOCKS = N_ITEMS // BLOCK_ITEMS
MAIN_ITEMS = NUM_BLOCKS * BLOCK_ITEMS
TAIL_ITEMS = N_ITEMS - MAIN_ITEMS

_LOG_EPS = math.log(1e-8)


def _merge(m_old, z_old, m_b, z_b):
    m_new = jnp.maximum(m_old, m_b)
    z_new = z_old * jnp.exp(m_old - m_new) + z_b * jnp.exp(m_b - m_new)
    return m_new, z_new


def _tc_score_body(e_ref, b_ref, et_ref, bt_ref, u_ref,
                   smain_ref, stail_ref, c_ref, m_ref, z_ref):
    i = pl.program_id(0)

    @pl.when(i == 0)
    def _():
        m_ref[...] = jnp.full((1, 1), -1e30, jnp.float32)
        z_ref[...] = jnp.zeros((1, 1), jnp.float32)
        c_ref[...] = jnp.zeros((1, 1), jnp.float32)
        stail_ref[...] = jnp.zeros((TAIL_ITEMS,), jnp.float32)

    ut = u_ref[...]
    s = jnp.sum(jnp.transpose(e_ref[...]) * ut, axis=0) + b_ref[...]
    smain_ref[...] = s
    m_blk = jnp.max(s)
    z_blk = jnp.sum(jnp.exp(s - m_blk))
    m_new, z_new = _merge(m_ref[...], z_ref[...],
                          jnp.full((1, 1), m_blk), jnp.full((1, 1), z_blk))
    m_ref[...] = m_new
    z_ref[...] = z_new

    @pl.when(i == NUM_BLOCKS - 1)
    def _():
        st = jnp.sum(jnp.transpose(et_ref[...]) * ut, axis=0) + bt_ref[...]
        stail_ref[...] = st
        mt = jnp.max(st)
        zt = jnp.sum(jnp.exp(st - mt))
        m_f, z_f = _merge(m_new, z_new,
                          jnp.full((1, 1), mt), jnp.full((1, 1), zt))
        c_ref[...] = m_f + jnp.log(z_f)


def _tc_scores(E, B, e_tail, b_tail, u):
    return pl.pallas_call(
        _tc_score_body,
        grid=(NUM_BLOCKS,),
        in_specs=[
            pl.BlockSpec((BLOCK_ITEMS, D_DIM), lambda i: (i, 0)),
            pl.BlockSpec((BLOCK_ITEMS,), lambda i: (i,)),
            pl.BlockSpec((TAIL_ITEMS, D_DIM), lambda i: (0, 0)),
            pl.BlockSpec((TAIL_ITEMS,), lambda i: (0,)),
            pl.BlockSpec((D_DIM, 1), lambda i: (0, 0)),
        ],
        out_specs=[
            pl.BlockSpec((BLOCK_ITEMS,), lambda i: (i,)),
            pl.BlockSpec((TAIL_ITEMS,), lambda i: (0,)),
            pl.BlockSpec((1, 1), lambda i: (0, 0)),
        ],
        out_shape=[
            jax.ShapeDtypeStruct((MAIN_ITEMS,), jnp.float32),
            jax.ShapeDtypeStruct((TAIL_ITEMS,), jnp.float32),
            jax.ShapeDtypeStruct((1, 1), jnp.float32),
        ],
        scratch_shapes=[
            pltpu.VMEM((1, 1), jnp.float32),
            pltpu.VMEM((1, 1), jnp.float32),
        ],
        compiler_params=pltpu.CompilerParams(
            dimension_semantics=("arbitrary",),
        ),
    )(E, B, e_tail, b_tail, u)


def _sc_gather(s_main, s_tail, idx):
    mesh = plsc.VectorSubcoreMesh(core_axis_name="c", subcore_axis_name="s")

    @functools.partial(
        pl.kernel,
        mesh=mesh,
        out_type=jax.ShapeDtypeStruct((S_SAMPLES,), jnp.float32),
        scratch_types=[
            pltpu.VMEM((ROWS_PER_WORKER,), jnp.int32),
            pltpu.VMEM((ROWS_PER_WORKER,), jnp.int32),
            pltpu.VMEM((ROWS_PER_WORKER,), jnp.int32),
            pltpu.VMEM((ROWS_PER_WORKER,), jnp.float32),
            pltpu.VMEM((ROWS_PER_WORKER,), jnp.float32),
            pltpu.VMEM((ROWS_PER_WORKER,), jnp.float32),
            pltpu.SemaphoreType.DMA,
            pltpu.SemaphoreType.DMA,
        ],
    )
    def gather_kernel(smain_hbm, stail_hbm, idx_hbm, out_hbm,
                      idx_v, im_v, it_v, gm_v, gt_v, out_v, sem_m, sem_t):
        wid = lax.axis_index("s") * 2 + lax.axis_index("c")
        base = wid * ROWS_PER_WORKER
        pltpu.sync_copy(idx_hbm.at[pl.ds(base, ROWS_PER_WORKER)], idx_v)
        for c in range(ROWS_PER_WORKER // LANES):
            sl = pl.ds(c * LANES, LANES)
            ix = idx_v[sl]
            im_v[sl] = jnp.minimum(ix, MAIN_ITEMS - 1)
            it_v[sl] = jnp.clip(ix - MAIN_ITEMS, 0, TAIL_ITEMS - 1)
        cp_m = pltpu.async_copy(smain_hbm.at[im_v], gm_v, sem_m)
        cp_t = pltpu.async_copy(stail_hbm.at[it_v], gt_v, sem_t)
        cp_m.wait()
        cp_t.wait()
        for c in range(ROWS_PER_WORKER // LANES):
            sl = pl.ds(c * LANES, LANES)
            out_v[sl] = jnp.where(idx_v[sl] < MAIN_ITEMS, gm_v[sl], gt_v[sl])
        pltpu.sync_copy(out_v, out_hbm.at[pl.ds(base, ROWS_PER_WORKER)])

    return gather_kernel(s_main, s_tail, idx)


def _tc_loss_body(s_ref, rew_ref, c_ref, out_ref):
    c = c_ref[...].reshape(())
    logp = jnp.maximum(s_ref[...] - c, _LOG_EPS)
    out_ref[...] = jnp.full((1, 1), -jnp.mean(logp * rew_ref[...]))


def _tc_loss(s_smp, reward, c):
    return pl.pallas_call(
        _tc_loss_body,
        out_shape=jax.ShapeDtypeStruct((1, 1), jnp.float32),
    )(s_smp, reward, c)


def kernel(G_user_embeddings, G_item_embeddings, G_item_bias, user_index,
           sample, reward):
    u = jnp.transpose(
        lax.dynamic_slice_in_dim(G_user_embeddings, user_index, 1, axis=0))
    idx = sample.astype(jnp.int32)

    e_tail = lax.slice(G_item_embeddings, (MAIN_ITEMS, 0), (N_ITEMS, D_DIM))
    b_tail = lax.slice(G_item_bias, (MAIN_ITEMS,), (N_ITEMS,))

    s_main, s_tail, c = _tc_scores(G_item_embeddings, G_item_bias,
                                   e_tail, b_tail, u)
    s_smp = _sc_gather(s_main, s_tail, idx)
    loss = _tc_loss(s_smp, reward, c)
    return loss.reshape(())
